# baseline (speedup 1.0000x reference)
import functools

import jax
import jax.numpy as jnp
from jax import lax
from jax.experimental import pallas as pl
from jax.experimental.pallas import tpu as pltpu
from jax.experimental.pallas import tpu_sc as plsc

_CLASS_NUM = 100000
_CHANNEL = 128
_SIZE = 4
_BATCH = 1024
_C = 20
_PAIRS = _BATCH * _C
_XDIM = _SIZE * _C * _SIZE


def _make_sc_gather():
    info = plsc.get_sparse_core_info()
    nw = info.num_cores * info.num_subcores
    rows_per_w = _PAIRS // nw
    k_chunks = rows_per_w // 128
    mesh = plsc.VectorSubcoreMesh(core_axis_name="c", subcore_axis_name="s")

    @functools.partial(
        pl.kernel,
        mesh=mesh,
        out_type=jax.ShapeDtypeStruct((_PAIRS, _CHANNEL), jnp.float32),
        scratch_types=[
            pltpu.VMEM((rows_per_w,), jnp.int32),
            pltpu.VMEM((rows_per_w, _CHANNEL), jnp.float32),
            pltpu.SemaphoreType.DMA,
        ],
    )
    def sc_gather(table_hbm, lab_hbm, out_hbm, idx_v, rows_v, sem):
        wid = lax.axis_index("s") * info.num_cores + lax.axis_index("c")
        pltpu.sync_copy(lab_hbm.at[pl.ds(wid * rows_per_w, rows_per_w)], idx_v)
        copies = []
        for kk in range(k_chunks):
            copies.append(
                pltpu.async_copy(
                    table_hbm.at[idx_v.at[pl.ds(kk * 128, 128)]],
                    rows_v.at[pl.ds(kk * 128, 128)],
                    sem,
                )
            )
        for cp in copies:
            cp.wait()
        pltpu.sync_copy(rows_v, out_hbm.at[pl.ds(wid * rows_per_w, rows_per_w)])

    return sc_gather


_SC_GATHER = _make_sc_gather()

_BBLK = 8


def _tc_expand_body(e_ref, out_ref):
    jj = lax.broadcasted_iota(jnp.int32, (_C, _XDIM), 0)
    xx = lax.broadcasted_iota(jnp.int32, (_C, _XDIM), 1)
    p = ((xx % (_C * _SIZE)) // _SIZE == jj).astype(jnp.float32)
    for b in range(_BBLK):
        out_ref[b] = lax.dot_general(
            e_ref[b], p, (((0,), (0,)), ((), ())),
            preferred_element_type=jnp.float32,
        )


def _tc_expand(e3):
    return pl.pallas_call(
        _tc_expand_body,
        grid=(_BATCH // _BBLK,),
        in_specs=[pl.BlockSpec((_BBLK, _C, _CHANNEL), lambda g: (g, 0, 0))],
        out_specs=pl.BlockSpec((_BBLK, _CHANNEL, _XDIM), lambda g: (g, 0, 0)),
        out_shape=jax.ShapeDtypeStruct((_BATCH, _CHANNEL, _XDIM), jnp.float32),
    )(e3)


def kernel(labels, TextEmbeddings):
    table = TextEmbeddings.reshape(_CLASS_NUM, _CHANNEL)
    lab_flat = labels.reshape(_PAIRS)
    e = _SC_GATHER(table, lab_flat)
    e3 = e.reshape(_BATCH, _C, _CHANNEL)
    out = _tc_expand(e3)
    return out.reshape(_BATCH, _CHANNEL, _SIZE, _C * _SIZE)


# speedup vs baseline: 4.0031x; 4.0031x over previous
import functools

import jax
import jax.numpy as jnp
from jax import lax
from jax.experimental import pallas as pl
from jax.experimental.pallas import tpu as pltpu
from jax.experimental.pallas import tpu_sc as plsc

_CLASS_NUM = 100000
_CHANNEL = 128
_SIZE = 4
_BATCH = 1024
_C = 20
_PAIRS = _BATCH * _C
_XROWS = _C * _SIZE


def _make_sc_select():
    info = plsc.get_sparse_core_info()
    nw = info.num_cores * info.num_subcores
    rows_per_w = _PAIRS // nw
    b_per_w = _BATCH // nw
    bc = 4
    n_chunks = b_per_w // bc
    crows = bc * _C
    mesh = plsc.VectorSubcoreMesh(core_axis_name="c", subcore_axis_name="s")

    @functools.partial(
        pl.kernel,
        mesh=mesh,
        out_type=jax.ShapeDtypeStruct((_BATCH, _SIZE, _XROWS, _CHANNEL),
                                      jnp.float32),
        scratch_types=[
            pltpu.VMEM((rows_per_w,), jnp.int32),
            pltpu.VMEM((crows, _CHANNEL), jnp.float32),
            pltpu.VMEM((2, bc, _XROWS, _CHANNEL), jnp.float32),
            pltpu.SemaphoreType.DMA,
            pltpu.SemaphoreType.DMA,
        ],
    )
    def sc_select(table_hbm, lab_hbm, out_hbm, idx_v, rows_v, exp_v, gsem, osem):
        wid = lax.axis_index("s") * info.num_cores + lax.axis_index("c")
        ib = wid * b_per_w
        pltpu.sync_copy(lab_hbm.at[pl.ds(wid * rows_per_w, rows_per_w)], idx_v)

        def out_copies(cc, buf):
            return [
                pltpu.make_async_copy(
                    exp_v.at[buf],
                    out_hbm.at[pl.ds(ib + cc * bc, bc), s],
                    osem,
                )
                for s in range(_SIZE)
            ]

        def chunk_body(cc, _):
            buf = lax.rem(cc, 2)

            @pl.when(cc >= 2)
            def _drain():
                for cp in out_copies(cc - 2, buf):
                    cp.wait()

            pltpu.async_copy(
                table_hbm.at[idx_v.at[pl.ds(cc * crows, crows)]],
                rows_v, gsem,
            ).wait()

            def expand_row(r, _):
                b2 = lax.div(r, _C)
                j = lax.rem(r, _C)
                for l in range(_CHANNEL // 16):
                    v = rows_v[r, pl.ds(l * 16, 16)]
                    for t in range(_SIZE):
                        exp_v[buf, b2, j * _SIZE + t, pl.ds(l * 16, 16)] = v
                return 0

            lax.fori_loop(0, crows, expand_row, 0, unroll=2)

            for cp in out_copies(cc, buf):
                cp.start()
            return 0

        lax.fori_loop(0, n_chunks, chunk_body, 0)

        for cc in (n_chunks - 2, n_chunks - 1):
            for cp in out_copies(cc, cc % 2):
                cp.wait()

    return sc_select


_SC_SELECT = _make_sc_select()


def kernel(labels, TextEmbeddings):
    table = TextEmbeddings.reshape(_CLASS_NUM, _CHANNEL)
    lab_flat = labels.reshape(_PAIRS)
    out4 = _SC_SELECT(table, lab_flat)
    return jnp.transpose(out4, (0, 3, 1, 2))


# speedup vs baseline: 4.1381x; 1.0337x over previous
import functools

import jax
import jax.numpy as jnp
from jax import lax
from jax.experimental import pallas as pl
from jax.experimental.pallas import tpu as pltpu
from jax.experimental.pallas import tpu_sc as plsc

_CLASS_NUM = 100000
_CHANNEL = 128
_SIZE = 4
_BATCH = 1024
_C = 20
_PAIRS = _BATCH * _C
_XROWS = _C * _SIZE


def _make_sc_select():
    info = plsc.get_sparse_core_info()
    nw = info.num_cores * info.num_subcores
    rows_per_w = _PAIRS // nw
    b_per_w = _BATCH // nw
    bc = 4
    n_chunks = b_per_w // bc
    crows = bc * _C
    mesh = plsc.VectorSubcoreMesh(core_axis_name="c", subcore_axis_name="s")

    @functools.partial(
        pl.kernel,
        mesh=mesh,
        out_type=jax.ShapeDtypeStruct((_BATCH, _SIZE, _XROWS, _CHANNEL),
                                      jnp.float32),
        scratch_types=[
            pltpu.VMEM((rows_per_w,), jnp.int32),
            pltpu.VMEM((2, crows, _CHANNEL), jnp.float32),
            pltpu.VMEM((2, bc, _XROWS, _CHANNEL), jnp.float32),
            pltpu.SemaphoreType.DMA,
            pltpu.SemaphoreType.DMA,
        ],
    )
    def sc_select(table_hbm, lab_hbm, out_hbm, idx_v, rows_v, exp_v, gsem, osem):
        wid = lax.axis_index("s") * info.num_cores + lax.axis_index("c")
        ib = wid * b_per_w
        pltpu.sync_copy(lab_hbm.at[pl.ds(wid * rows_per_w, rows_per_w)], idx_v)

        def gather(cc):
            return pltpu.make_async_copy(
                table_hbm.at[idx_v.at[pl.ds(cc * crows, crows)]],
                rows_v.at[lax.rem(cc, 2)],
                gsem,
            )

        def out_copies(cc, buf):
            return [
                pltpu.make_async_copy(
                    exp_v.at[buf],
                    out_hbm.at[pl.ds(ib + cc * bc, bc), s],
                    osem,
                )
                for s in range(_SIZE)
            ]

        gather(0).start()

        def chunk_body(cc, _):
            buf = lax.rem(cc, 2)

            @pl.when(cc + 1 < n_chunks)
            def _prefetch():
                gather(cc + 1).start()

            gather(cc).wait()

            @pl.when(cc >= 2)
            def _drain():
                for cp in out_copies(cc - 2, buf):
                    cp.wait()

            def expand_row(r, _):
                b2 = lax.div(r, _C)
                j = lax.rem(r, _C)
                for l in range(_CHANNEL // 16):
                    v = rows_v[buf, r, pl.ds(l * 16, 16)]
                    for t in range(_SIZE):
                        exp_v[buf, b2, j * _SIZE + t, pl.ds(l * 16, 16)] = v
                return 0

            lax.fori_loop(0, crows, expand_row, 0, unroll=2)

            for cp in out_copies(cc, buf):
                cp.start()
            return 0

        lax.fori_loop(0, n_chunks, chunk_body, 0)

        for cc in (n_chunks - 2, n_chunks - 1):
            for cp in out_copies(cc, cc % 2):
                cp.wait()

    return sc_select


_SC_SELECT = _make_sc_select()


def kernel(labels, TextEmbeddings):
    table = TextEmbeddings.reshape(_CLASS_NUM, _CHANNEL)
    lab_flat = labels.reshape(_PAIRS)
    out4 = _SC_SELECT(table, lab_flat)
    return jnp.transpose(out4, (0, 3, 1, 2))
